# Initial kernel scaffold; baseline (speedup 1.0000x reference)
import functools

import jax
import jax.numpy as jnp
from jax import lax
from jax.experimental import pallas as pl
from jax.experimental.pallas import tpu as pltpu
from jax.experimental.pallas import tpu_sc as plsc

N = 10000
E = 320000
F_IN = 128
HID = 128
M = 13
COMP = 6
CLS = 7

NC = 2
NS = 16
NW = NC * NS
K = 80
CHUNKS_TOT = E // K
CPW = CHUNKS_TOT // NW
RPS = N // NS
ZR = 125


def _sc_segment_sums(h, src2d, dst2d, w3d):
    mesh = plsc.VectorSubcoreMesh(core_axis_name="c", subcore_axis_name="s")

    @functools.partial(
        pl.kernel,
        out_type=jax.ShapeDtypeStruct((NC, M, N, HID), jnp.float32),
        mesh=mesh,
        scratch_types=[
            pltpu.VMEM((CPW, K), jnp.int32),
            pltpu.VMEM((CPW, K), jnp.int32),
            pltpu.VMEM((CPW, K), jnp.float32),
            pltpu.VMEM((K, HID), jnp.float32),
            pltpu.VMEM((ZR, HID), jnp.float32),
            pltpu.VMEM_SHARED((N, HID), jnp.float32),
        ],
    )
    def body(h_hbm, src_hbm, dst_hbm, w_hbm, out_hbm,
             src_v, dst_v, w_v, rows_v, zeros_v, acc):
        cid = lax.axis_index("c")
        sid = lax.axis_index("s")
        wid = cid * NS + sid

        pltpu.sync_copy(src_hbm.at[pl.ds(wid * CPW, CPW)], src_v)
        pltpu.sync_copy(dst_hbm.at[pl.ds(wid * CPW, CPW)], dst_v)

        def zero_row(r, carry):
            for b in range(HID // 16):
                zeros_v[r, pl.ds(b * 16, 16)] = jnp.zeros((16,), jnp.float32)
            return carry

        lax.fori_loop(0, ZR, zero_row, 0)

        def motif_body(m, carry):
            pltpu.sync_copy(w_hbm.at[m, pl.ds(wid * CPW, CPW)], w_v)
            for z in range(RPS // ZR):
                pltpu.sync_copy(zeros_v, acc.at[pl.ds(sid * RPS + z * ZR, ZR)])
            plsc.subcore_barrier()

            def chunk_body(j, carry2):
                pltpu.sync_copy(h_hbm.at[src_v.at[j]], rows_v)

                def edge_body(i, carry3):
                    ws = w_v[j, i]
                    for b in range(HID // 16):
                        sl = pl.ds(b * 16, 16)
                        rows_v[i, sl] = rows_v[i, sl] * ws
                    return carry3

                lax.fori_loop(0, K, edge_body, 0, unroll=2)
                pltpu.sync_copy(rows_v, acc.at[dst_v.at[j]], add=True)
                return carry2

            lax.fori_loop(0, CPW, chunk_body, 0)
            plsc.subcore_barrier()
            for z in range(RPS // ZR):
                sl = pl.ds(sid * RPS + z * ZR, ZR)
                pltpu.sync_copy(acc.at[sl], out_hbm.at[cid, m, sl])
            return carry

        lax.fori_loop(0, M, motif_body, 0)

    return body(h, src2d, dst2d, w3d)


NB = 500


def _tc_dense_body(gp_ref, w_ref, att_ref, c_ref, dw_ref, db_ref, out_ref):
    outs = []
    for m in range(M):
        g = gp_ref[0, m] + gp_ref[1, m]
        z = jnp.dot(g, w_ref[m], preferred_element_type=jnp.float32)
        a = jax.nn.sigmoid(
            jnp.dot(z, att_ref[m].reshape(HID, 1),
                    preferred_element_type=jnp.float32))
        outs.append(jnp.dot(z * a, c_ref[m],
                            preferred_element_type=jnp.float32))
    hcat = jax.nn.relu(jnp.concatenate(outs, axis=1))
    logits = jnp.dot(hcat, dw_ref[...],
                     preferred_element_type=jnp.float32) + db_ref[...]
    mx = jnp.max(logits, axis=1, keepdims=True)
    lse = mx + jnp.log(jnp.sum(jnp.exp(logits - mx), axis=1, keepdims=True))
    out_ref[...] = logits - lse


def _tc_dense(gp, W, att, C, dense_W, dense_b):
    return pl.pallas_call(
        _tc_dense_body,
        grid=(N // NB,),
        in_specs=[
            pl.BlockSpec((NC, M, NB, HID), lambda i: (0, 0, i, 0)),
            pl.BlockSpec((M, HID, HID), lambda i: (0, 0, 0)),
            pl.BlockSpec((M, HID), lambda i: (0, 0)),
            pl.BlockSpec((M, HID, COMP), lambda i: (0, 0, 0)),
            pl.BlockSpec((M * COMP, CLS), lambda i: (0, 0)),
            pl.BlockSpec((CLS,), lambda i: (0,)),
        ],
        out_specs=pl.BlockSpec((NB, CLS), lambda i: (i, 0)),
        out_shape=jax.ShapeDtypeStruct((N, CLS), jnp.float32),
    )(gp, W, att, C, dense_W, dense_b)


def kernel(h, edge_index, motif_w, W, att, C, dense_W, dense_b):
    src2d = edge_index[0].reshape(CHUNKS_TOT, K)
    dst2d = edge_index[1].reshape(CHUNKS_TOT, K)
    w3d = motif_w.reshape(M, CHUNKS_TOT, K)
    gp = _sc_segment_sums(h, src2d, dst2d, w3d)
    return _tc_dense(gp, W, att, C, dense_W, dense_b)


# speedup vs baseline: 1.3050x; 1.3050x over previous
import functools

import jax
import jax.numpy as jnp
from jax import lax
from jax.experimental import pallas as pl
from jax.experimental.pallas import tpu as pltpu
from jax.experimental.pallas import tpu_sc as plsc

N = 10000
E = 320000
F_IN = 128
HID = 128
M = 13
COMP = 6
CLS = 7

NC = 2
NS = 16
K = 80
CHUNKS_TOT = E // K
CPW = CHUNKS_TOT // NS
HH = HID // NC
ACH = 1000
ZR = 200


def _sc_segment_sums(h, src3d, dst3d, w4d):
    mesh = plsc.VectorSubcoreMesh(core_axis_name="c", subcore_axis_name="s")

    @functools.partial(
        pl.kernel,
        out_type=jax.ShapeDtypeStruct((NC, M, N, HH), jnp.float32),
        mesh=mesh,
        scratch_types=[
            pltpu.VMEM((CPW, K), jnp.int32),
            pltpu.VMEM((CPW, K), jnp.int32),
            pltpu.VMEM((CPW, K), jnp.float32),
            pltpu.VMEM((K, HID), jnp.float32),
            pltpu.VMEM((K, HH), jnp.float32),
            pltpu.VMEM((ZR, HH), jnp.float32),
            pltpu.VMEM_SHARED((N, HH), jnp.float32),
        ],
        compiler_params=pltpu.CompilerParams(use_tc_tiling_on_sc=False),
    )
    def body(h_hbm, src_hbm, dst_hbm, w_hbm, out_hbm,
             src_v, dst_v, w_v, rows_v, sc_v, zeros_v, acc):
        cid = lax.axis_index("c")
        sid = lax.axis_index("s")
        col0 = cid * HH

        pltpu.sync_copy(src_hbm.at[sid], src_v)
        pltpu.sync_copy(dst_hbm.at[sid], dst_v)

        def zero_row(r, carry):
            for b in range(HH // 16):
                zeros_v[r, pl.ds(b * 16, 16)] = jnp.zeros((16,), jnp.float32)
            return carry

        lax.fori_loop(0, ZR, zero_row, 0)

        def motif_body(m, carry):
            pltpu.sync_copy(w_hbm.at[m, sid], w_v)
            @pl.when(sid < N // ACH)
            def _():
                for z in range(ACH // ZR):
                    pltpu.sync_copy(zeros_v,
                                    acc.at[pl.ds(sid * ACH + z * ZR, ZR)])
            plsc.subcore_barrier()

            def chunk_body(j, carry2):
                pltpu.sync_copy(h_hbm.at[src_v.at[j]], rows_v)

                def grp_body(g, carry3):
                    wv = w_v[j, pl.ds(g * 16, 16)]
                    for e in range(16):
                        ws = wv[e]
                        row = g * 16 + e
                        for b in range(HH // 16):
                            sc_v[row, pl.ds(b * 16, 16)] = (
                                rows_v[row, pl.ds(col0 + b * 16, 16)] * ws)
                    return carry3

                lax.fori_loop(0, K // 16, grp_body, 0)
                pltpu.sync_copy(sc_v, acc.at[dst_v.at[j]], add=True)
                return carry2

            lax.fori_loop(0, CPW, chunk_body, 0)
            plsc.subcore_barrier()

            @pl.when(sid < N // ACH)
            def _():
                sl = pl.ds(sid * ACH, ACH)
                pltpu.sync_copy(acc.at[sl], out_hbm.at[cid, m, sl])
            return carry

        lax.fori_loop(0, M, motif_body, 0)

    return body(h, src3d, dst3d, w4d)


NB = 1000


def _tc_dense_body(gp_ref, w_ref, att_ref, c_ref, dw_ref, db_ref, out_ref):
    outs = []
    for m in range(M):
        g = jnp.concatenate([gp_ref[0, m], gp_ref[1, m]], axis=1)
        z = jnp.dot(g, w_ref[m], preferred_element_type=jnp.float32)
        a = jax.nn.sigmoid(
            jnp.dot(z, att_ref[m].reshape(HID, 1),
                    preferred_element_type=jnp.float32))
        outs.append(jnp.dot(z * a, c_ref[m],
                            preferred_element_type=jnp.float32))
    hcat = jax.nn.relu(jnp.concatenate(outs, axis=1))
    logits = jnp.dot(hcat, dw_ref[...],
                     preferred_element_type=jnp.float32) + db_ref[...]
    mx = jnp.max(logits, axis=1, keepdims=True)
    lse = mx + jnp.log(jnp.sum(jnp.exp(logits - mx), axis=1, keepdims=True))
    out_ref[...] = logits - lse


def _tc_dense(gp, W, att, C, dense_W, dense_b):
    return pl.pallas_call(
        _tc_dense_body,
        grid=(N // NB,),
        in_specs=[
            pl.BlockSpec((NC, M, NB, HH), lambda i: (0, 0, i, 0)),
            pl.BlockSpec((M, HID, HID), lambda i: (0, 0, 0)),
            pl.BlockSpec((M, HID), lambda i: (0, 0)),
            pl.BlockSpec((M, HID, COMP), lambda i: (0, 0, 0)),
            pl.BlockSpec((M * COMP, CLS), lambda i: (0, 0)),
            pl.BlockSpec((CLS,), lambda i: (0,)),
        ],
        out_specs=pl.BlockSpec((NB, CLS), lambda i: (i, 0)),
        out_shape=jax.ShapeDtypeStruct((N, CLS), jnp.float32),
    )(gp, W, att, C, dense_W, dense_b)


def kernel(h, edge_index, motif_w, W, att, C, dense_W, dense_b):
    src3d = edge_index[0].reshape(NS, CPW, K)
    dst3d = edge_index[1].reshape(NS, CPW, K)
    w4d = motif_w.reshape(M, NS, CPW, K)
    gp = _sc_segment_sums(h, src3d, dst3d, w4d)
    return _tc_dense(gp, W, att, C, dense_W, dense_b)


# speedup vs baseline: 5.2556x; 4.0271x over previous
import functools

import jax
import jax.numpy as jnp
from jax import lax
from jax.experimental import pallas as pl
from jax.experimental.pallas import tpu as pltpu
from jax.experimental.pallas import tpu_sc as plsc

N = 10000
E = 320000
F_IN = 128
HID = 128
M = 13
COMP = 6
CLS = 7

NC = 2
NS = 16
NW = NC * NS
K = 80
CPW = E // K // NW
HH = HID // 2
ACH = 1000
ZR = 200


def _sc_segment_sums(h_lo, h_hi, src3d, dst3d, w4d):
    mesh = plsc.VectorSubcoreMesh(core_axis_name="c", subcore_axis_name="s")

    @functools.partial(
        pl.kernel,
        out_type=jax.ShapeDtypeStruct((2, NC, M, N, HH), jnp.float32),
        mesh=mesh,
        scratch_types=[
            pltpu.VMEM((CPW, K), jnp.int32),
            pltpu.VMEM((CPW, K), jnp.int32),
            pltpu.VMEM((CPW, K), jnp.float32),
            pltpu.VMEM((K, HH), jnp.float32),
            pltpu.VMEM((K, HH), jnp.float32),
            pltpu.VMEM((K, HH), jnp.float32),
            pltpu.VMEM((K, HH), jnp.float32),
            pltpu.VMEM((ZR, HH), jnp.float32),
            pltpu.VMEM_SHARED((N, HH), jnp.float32),
            pltpu.SemaphoreType.DMA,
            pltpu.SemaphoreType.DMA,
            pltpu.SemaphoreType.DMA,
            pltpu.SemaphoreType.DMA,
        ],
        compiler_params=pltpu.CompilerParams(use_tc_tiling_on_sc=False),
    )
    def body(hlo_hbm, hhi_hbm, src_hbm, dst_hbm, w_hbm, out_hbm,
             src_v, dst_v, w_v, rows0, rows1, sc0, sc1, zeros_v, acc,
             g0, g1, s0, s1):
        cid = lax.axis_index("c")
        sid = lax.axis_index("s")
        wid = cid * NS + sid

        pltpu.sync_copy(src_hbm.at[wid], src_v)
        pltpu.sync_copy(dst_hbm.at[wid], dst_v)

        def zero_row(r, carry):
            for b in range(HH // 16):
                zeros_v[r, pl.ds(b * 16, 16)] = jnp.zeros((16,), jnp.float32)
            return carry

        lax.fori_loop(0, ZR, zero_row, 0)

        def scale(rows, scb, j):
            def grp_body(g, carry3):
                wv = w_v[j, pl.ds(g * 16, 16)]
                for e in range(16):
                    ws = wv[e]
                    row = g * 16 + e
                    for b in range(HH // 16):
                        sl = pl.ds(b * 16, 16)
                        scb[row, sl] = rows[row, sl] * ws
                return carry3
            lax.fori_loop(0, K // 16, grp_body, 0)

        def motif_body(m, carry):
            pltpu.sync_copy(w_hbm.at[m, wid], w_v)
            for hf, h_hbm in enumerate((hlo_hbm, hhi_hbm)):
                @pl.when(sid < N // ACH)
                def _():
                    for z in range(ACH // ZR):
                        pltpu.sync_copy(
                            zeros_v, acc.at[pl.ds(sid * ACH + z * ZR, ZR)])
                plsc.subcore_barrier()

                pltpu.async_copy(h_hbm.at[src_v.at[0]], rows0, g0)
                pltpu.async_copy(h_hbm.at[src_v.at[1]], rows1, g1)

                def step(j, jj, rows, scb, gsem, ssem):
                    pltpu.make_async_copy(h_hbm.at[src_v.at[j]], rows,
                                          gsem).wait()

                    @pl.when(jj > 0)
                    def _():
                        pltpu.make_async_copy(scb, acc.at[dst_v.at[j]],
                                              ssem).wait()

                    scale(rows, scb, j)

                    @pl.when(j + 2 < CPW)
                    def _():
                        pltpu.async_copy(h_hbm.at[src_v.at[j + 2]], rows,
                                         gsem)

                    pltpu.async_copy(scb, acc.at[dst_v.at[j]], ssem,
                                     add=True)

                def pair_body(jj, carry2):
                    step(2 * jj, jj, rows0, sc0, g0, s0)
                    step(2 * jj + 1, jj, rows1, sc1, g1, s1)
                    return carry2

                lax.fori_loop(0, CPW // 2, pair_body, 0)

                if CPW % 2:
                    j = CPW - 1
                    pltpu.make_async_copy(h_hbm.at[src_v.at[j]], rows0,
                                          g0).wait()
                    pltpu.make_async_copy(sc0, acc.at[dst_v.at[j]],
                                          s0).wait()
                    scale(rows0, sc0, j)
                    pltpu.async_copy(sc0, acc.at[dst_v.at[j]], s0, add=True)

                pltpu.make_async_copy(sc0, acc.at[dst_v.at[CPW - 1]],
                                      s0).wait()
                pltpu.make_async_copy(sc1, acc.at[dst_v.at[CPW - 2]],
                                      s1).wait()
                plsc.subcore_barrier()

                @pl.when(sid < N // ACH)
                def _():
                    sl = pl.ds(sid * ACH, ACH)
                    pltpu.sync_copy(acc.at[sl], out_hbm.at[hf, cid, m, sl])
            return carry

        lax.fori_loop(0, M, motif_body, 0)

    return body(h_lo, h_hi, src3d, dst3d, w4d)


NB = 400


def _tc_dense_body(gp_ref, w_ref, att_ref, c_ref, dw_ref, db_ref, out_ref):
    outs = []
    for m in range(M):
        g = jnp.concatenate(
            [gp_ref[0, 0, m] + gp_ref[0, 1, m],
             gp_ref[1, 0, m] + gp_ref[1, 1, m]], axis=1)
        z = jnp.dot(g, w_ref[m], preferred_element_type=jnp.float32)
        a = jax.nn.sigmoid(
            jnp.dot(z, att_ref[m].reshape(HID, 1),
                    preferred_element_type=jnp.float32))
        outs.append(jnp.dot(z * a, c_ref[m],
                            preferred_element_type=jnp.float32))
    hcat = jax.nn.relu(jnp.concatenate(outs, axis=1))
    logits = jnp.dot(hcat, dw_ref[...],
                     preferred_element_type=jnp.float32) + db_ref[...]
    mx = jnp.max(logits, axis=1, keepdims=True)
    lse = mx + jnp.log(jnp.sum(jnp.exp(logits - mx), axis=1, keepdims=True))
    out_ref[...] = logits - lse


def _tc_dense(gp, W, att, C, dense_W, dense_b):
    return pl.pallas_call(
        _tc_dense_body,
        grid=(N // NB,),
        in_specs=[
            pl.BlockSpec((2, NC, M, NB, HH), lambda i: (0, 0, 0, i, 0)),
            pl.BlockSpec((M, HID, HID), lambda i: (0, 0, 0)),
            pl.BlockSpec((M, HID), lambda i: (0, 0)),
            pl.BlockSpec((M, HID, COMP), lambda i: (0, 0, 0)),
            pl.BlockSpec((M * COMP, CLS), lambda i: (0, 0)),
            pl.BlockSpec((CLS,), lambda i: (0,)),
        ],
        out_specs=pl.BlockSpec((NB, CLS), lambda i: (i, 0)),
        out_shape=jax.ShapeDtypeStruct((N, CLS), jnp.float32),
    )(gp, W, att, C, dense_W, dense_b)


def kernel(h, edge_index, motif_w, W, att, C, dense_W, dense_b):
    src3d = edge_index[0].reshape(NW, CPW, K)
    dst3d = edge_index[1].reshape(NW, CPW, K)
    w4d = motif_w.reshape(M, NW, CPW, K)
    gp = _sc_segment_sums(h[:, :HH], h[:, HH:], src3d, dst3d, w4d)
    return _tc_dense(gp, W, att, C, dense_W, dense_b)


# speedup vs baseline: 5.2852x; 1.0056x over previous
import functools

import jax
import jax.numpy as jnp
from jax import lax
from jax.experimental import pallas as pl
from jax.experimental.pallas import tpu as pltpu
from jax.experimental.pallas import tpu_sc as plsc

N = 10000
E = 320000
F_IN = 128
HID = 128
M = 13
COMP = 6
CLS = 7

NC = 2
NS = 16
NW = NC * NS
K = 80
CPW = E // K // NW
HH = HID // 2
ACH = 1000
ZR = 200


def _sc_segment_sums(h_lo, h_hi, src3d, dst3d, w4d):
    mesh = plsc.VectorSubcoreMesh(core_axis_name="c", subcore_axis_name="s")

    @functools.partial(
        pl.kernel,
        out_type=jax.ShapeDtypeStruct((2, NC, M, N, HH), jnp.float32),
        mesh=mesh,
        scratch_types=[
            pltpu.VMEM((CPW, K), jnp.int32),
            pltpu.VMEM((CPW, K), jnp.int32),
            pltpu.VMEM((CPW, K), jnp.float32),
            pltpu.VMEM((K, HH), jnp.float32),
            pltpu.VMEM((K, HH), jnp.float32),
            pltpu.VMEM((K, HH), jnp.float32),
            pltpu.VMEM((K, HH), jnp.float32),
            pltpu.VMEM((ZR, HH), jnp.float32),
            pltpu.VMEM_SHARED((N, HH), jnp.float32),
            pltpu.SemaphoreType.DMA,
            pltpu.SemaphoreType.DMA,
            pltpu.SemaphoreType.DMA,
            pltpu.SemaphoreType.DMA,
        ],
        compiler_params=pltpu.CompilerParams(use_tc_tiling_on_sc=False),
    )
    def body(hlo_hbm, hhi_hbm, src_hbm, dst_hbm, w_hbm, out_hbm,
             src_v, dst_v, w_v, rows0, rows1, sc0, sc1, zeros_v, acc,
             g0, g1, s0, s1):
        cid = lax.axis_index("c")
        sid = lax.axis_index("s")
        wid = cid * NS + sid

        pltpu.sync_copy(src_hbm.at[wid], src_v)
        pltpu.sync_copy(dst_hbm.at[wid], dst_v)

        def zero_row(r, carry):
            for b in range(HH // 16):
                zeros_v[r, pl.ds(b * 16, 16)] = jnp.zeros((16,), jnp.float32)
            return carry

        lax.fori_loop(0, ZR, zero_row, 0)

        def scale(rows, scb, j):
            for g in range(K // 16):
                wv = w_v[j, pl.ds(g * 16, 16)]
                for e in range(16):
                    ws = wv[e]
                    row = g * 16 + e
                    for b in range(HH // 16):
                        sl = pl.ds(b * 16, 16)
                        scb[row, sl] = rows[row, sl] * ws

        def motif_body(m, carry):
            pltpu.sync_copy(w_hbm.at[m, wid], w_v)
            for hf, h_hbm in enumerate((hlo_hbm, hhi_hbm)):
                @pl.when(sid < N // ACH)
                def _():
                    for z in range(ACH // ZR):
                        pltpu.sync_copy(
                            zeros_v, acc.at[pl.ds(sid * ACH + z * ZR, ZR)])
                plsc.subcore_barrier()

                pltpu.async_copy(h_hbm.at[src_v.at[0]], rows0, g0)
                pltpu.async_copy(h_hbm.at[src_v.at[1]], rows1, g1)

                def step(j, jj, rows, scb, gsem, ssem):
                    pltpu.make_async_copy(h_hbm.at[src_v.at[j]], rows,
                                          gsem).wait()

                    @pl.when(jj > 0)
                    def _():
                        pltpu.make_async_copy(scb, acc.at[dst_v.at[j]],
                                              ssem).wait()

                    scale(rows, scb, j)

                    @pl.when(j + 2 < CPW)
                    def _():
                        pltpu.async_copy(h_hbm.at[src_v.at[j + 2]], rows,
                                         gsem)

                    pltpu.async_copy(scb, acc.at[dst_v.at[j]], ssem,
                                     add=True)

                def pair_body(jj, carry2):
                    step(2 * jj, jj, rows0, sc0, g0, s0)
                    step(2 * jj + 1, jj, rows1, sc1, g1, s1)
                    return carry2

                lax.fori_loop(0, CPW // 2, pair_body, 0)

                if CPW % 2:
                    j = CPW - 1
                    pltpu.make_async_copy(h_hbm.at[src_v.at[j]], rows0,
                                          g0).wait()
                    pltpu.make_async_copy(sc0, acc.at[dst_v.at[j]],
                                          s0).wait()
                    scale(rows0, sc0, j)
                    pltpu.async_copy(sc0, acc.at[dst_v.at[j]], s0, add=True)

                pltpu.make_async_copy(sc0, acc.at[dst_v.at[CPW - 1]],
                                      s0).wait()
                pltpu.make_async_copy(sc1, acc.at[dst_v.at[CPW - 2]],
                                      s1).wait()
                plsc.subcore_barrier()

                @pl.when(sid < N // ACH)
                def _():
                    sl = pl.ds(sid * ACH, ACH)
                    pltpu.sync_copy(acc.at[sl], out_hbm.at[hf, cid, m, sl])
            return carry

        lax.fori_loop(0, M, motif_body, 0)

    return body(h_lo, h_hi, src3d, dst3d, w4d)


NB = 400


def _tc_dense_body(gp_ref, w_ref, att_ref, c_ref, dw_ref, db_ref, out_ref):
    outs = []
    for m in range(M):
        g = jnp.concatenate(
            [gp_ref[0, 0, m] + gp_ref[0, 1, m],
             gp_ref[1, 0, m] + gp_ref[1, 1, m]], axis=1)
        z = jnp.dot(g, w_ref[m], preferred_element_type=jnp.float32)
        a = jax.nn.sigmoid(
            jnp.dot(z, att_ref[m].reshape(HID, 1),
                    preferred_element_type=jnp.float32))
        outs.append(jnp.dot(z * a, c_ref[m],
                            preferred_element_type=jnp.float32))
    hcat = jax.nn.relu(jnp.concatenate(outs, axis=1))
    logits = jnp.dot(hcat, dw_ref[...],
                     preferred_element_type=jnp.float32) + db_ref[...]
    mx = jnp.max(logits, axis=1, keepdims=True)
    lse = mx + jnp.log(jnp.sum(jnp.exp(logits - mx), axis=1, keepdims=True))
    out_ref[...] = logits - lse


def _tc_dense(gp, W, att, C, dense_W, dense_b):
    return pl.pallas_call(
        _tc_dense_body,
        grid=(N // NB,),
        in_specs=[
            pl.BlockSpec((2, NC, M, NB, HH), lambda i: (0, 0, 0, i, 0)),
            pl.BlockSpec((M, HID, HID), lambda i: (0, 0, 0)),
            pl.BlockSpec((M, HID), lambda i: (0, 0)),
            pl.BlockSpec((M, HID, COMP), lambda i: (0, 0, 0)),
            pl.BlockSpec((M * COMP, CLS), lambda i: (0, 0)),
            pl.BlockSpec((CLS,), lambda i: (0,)),
        ],
        out_specs=pl.BlockSpec((NB, CLS), lambda i: (i, 0)),
        out_shape=jax.ShapeDtypeStruct((N, CLS), jnp.float32),
    )(gp, W, att, C, dense_W, dense_b)


def kernel(h, edge_index, motif_w, W, att, C, dense_W, dense_b):
    src3d = edge_index[0].reshape(NW, CPW, K)
    dst3d = edge_index[1].reshape(NW, CPW, K)
    w4d = motif_w.reshape(M, NW, CPW, K)
    gp = _sc_segment_sums(h[:, :HH], h[:, HH:], src3d, dst3d, w4d)
    return _tc_dense(gp, W, att, C, dense_W, dense_b)


# speedup vs baseline: 6.5821x; 1.2454x over previous
import functools

import jax
import jax.numpy as jnp
from jax import lax
from jax.experimental import pallas as pl
from jax.experimental.pallas import tpu as pltpu
from jax.experimental.pallas import tpu_sc as plsc

N = 10000
E = 320000
F_IN = 128
HID = 128
M = 13
COMP = 6
CLS = 7

NC = 2
NS = 16
NW = NC * NS
K = 80
CPW = E // K // NW
HH = HID // 2
ACH = 1000
ZR = 200


def _sc_segment_sums(h_lo, h_hi, src3d, dst3d, w4d):
    mesh = plsc.VectorSubcoreMesh(core_axis_name="c", subcore_axis_name="s")

    @functools.partial(
        pl.kernel,
        out_type=jax.ShapeDtypeStruct((2, NC, M, N, HH), jnp.float32),
        mesh=mesh,
        scratch_types=[
            pltpu.VMEM((CPW, K), jnp.int32),
            pltpu.VMEM((CPW, K), jnp.int32),
            pltpu.VMEM((CPW, K), jnp.float32),
            pltpu.VMEM((K, HH), jnp.float32),
            pltpu.VMEM((K, HH), jnp.float32),
            pltpu.VMEM((K, HH), jnp.float32),
            pltpu.VMEM((K, HH), jnp.float32),
            pltpu.VMEM((K, HH), jnp.float32),
            pltpu.VMEM((K, HH), jnp.float32),
            pltpu.VMEM((K, HH), jnp.float32),
            pltpu.VMEM((K, HH), jnp.float32),
            pltpu.VMEM((ZR, HH), jnp.float32),
            pltpu.VMEM_SHARED((N, HH), jnp.float32),
            pltpu.SemaphoreType.DMA,
            pltpu.SemaphoreType.DMA,
            pltpu.SemaphoreType.DMA,
            pltpu.SemaphoreType.DMA,
            pltpu.SemaphoreType.DMA,
            pltpu.SemaphoreType.DMA,
            pltpu.SemaphoreType.DMA,
            pltpu.SemaphoreType.DMA,
        ],
        compiler_params=pltpu.CompilerParams(use_tc_tiling_on_sc=False),
    )
    def body(hlo_hbm, hhi_hbm, src_hbm, dst_hbm, w_hbm, out_hbm,
             src_v, dst_v, w_v, rows0, rows1, rows2, rows3,
             sc0, sc1, sc2, sc3, zeros_v, acc,
             g0, g1, g2, g3, s0, s1, s2, s3):
        rows_bufs = (rows0, rows1, rows2, rows3)
        sc_bufs = (sc0, sc1, sc2, sc3)
        gsems = (g0, g1, g2, g3)
        ssems = (s0, s1, s2, s3)
        NBUF = 4
        cid = lax.axis_index("c")
        sid = lax.axis_index("s")
        wid = cid * NS + sid

        pltpu.sync_copy(src_hbm.at[wid], src_v)
        pltpu.sync_copy(dst_hbm.at[wid], dst_v)

        def zero_row(r, carry):
            for b in range(HH // 16):
                zeros_v[r, pl.ds(b * 16, 16)] = jnp.zeros((16,), jnp.float32)
            return carry

        lax.fori_loop(0, ZR, zero_row, 0)

        def scale(rows, scb, j):
            def grp_body(g, carry3):
                wv = w_v[j, pl.ds(g * 16, 16)]
                for e in range(16):
                    ws = wv[e]
                    row = g * 16 + e
                    for b in range(HH // 16):
                        sl = pl.ds(b * 16, 16)
                        scb[row, sl] = rows[row, sl] * ws
                return carry3
            lax.fori_loop(0, K // 16, grp_body, 0)

        def motif_body(m, carry):
            pltpu.sync_copy(w_hbm.at[m, wid], w_v)
            for hf, h_hbm in enumerate((hlo_hbm, hhi_hbm)):
                @pl.when(sid < N // ACH)
                def _():
                    for z in range(ACH // ZR):
                        pltpu.sync_copy(
                            zeros_v, acc.at[pl.ds(sid * ACH + z * ZR, ZR)])
                plsc.subcore_barrier()

                for b in range(NBUF):
                    pltpu.async_copy(h_hbm.at[src_v.at[b]], rows_bufs[b],
                                     gsems[b])

                def step(j, jj, bi):
                    rows, scb = rows_bufs[bi], sc_bufs[bi]
                    pltpu.make_async_copy(h_hbm.at[src_v.at[j]], rows,
                                          gsems[bi]).wait()

                    @pl.when(jj > 0)
                    def _():
                        pltpu.make_async_copy(scb, acc.at[dst_v.at[j]],
                                              ssems[bi]).wait()

                    scale(rows, scb, j)

                    @pl.when(j + NBUF < CPW)
                    def _():
                        pltpu.async_copy(h_hbm.at[src_v.at[j + NBUF]], rows,
                                         gsems[bi])

                    pltpu.async_copy(scb, acc.at[dst_v.at[j]], ssems[bi],
                                     add=True)

                def quad_body(jj, carry2):
                    for b in range(NBUF):
                        step(NBUF * jj + b, jj, b)
                    return carry2

                lax.fori_loop(0, CPW // NBUF, quad_body, 0)

                for t in range(CPW % NBUF):
                    j = (CPW // NBUF) * NBUF + t
                    rows, scb = rows_bufs[t], sc_bufs[t]
                    pltpu.make_async_copy(h_hbm.at[src_v.at[j]], rows,
                                          gsems[t]).wait()
                    pltpu.make_async_copy(scb, acc.at[dst_v.at[j]],
                                          ssems[t]).wait()
                    scale(rows, scb, j)
                    pltpu.async_copy(scb, acc.at[dst_v.at[j]], ssems[t],
                                     add=True)

                for b in range(NBUF):
                    pltpu.make_async_copy(sc_bufs[b], acc.at[dst_v.at[b]],
                                          ssems[b]).wait()
                plsc.subcore_barrier()

                @pl.when(sid < N // ACH)
                def _():
                    sl = pl.ds(sid * ACH, ACH)
                    pltpu.sync_copy(acc.at[sl], out_hbm.at[hf, cid, m, sl])
            return carry

        lax.fori_loop(0, M, motif_body, 0)

    return body(h_lo, h_hi, src3d, dst3d, w4d)


NB = 400


def _tc_dense_body(gp_ref, w_ref, att_ref, c_ref, dw_ref, db_ref, out_ref):
    outs = []
    for m in range(M):
        g = jnp.concatenate(
            [gp_ref[0, 0, m] + gp_ref[0, 1, m],
             gp_ref[1, 0, m] + gp_ref[1, 1, m]], axis=1)
        z = jnp.dot(g, w_ref[m], preferred_element_type=jnp.float32)
        a = jax.nn.sigmoid(
            jnp.dot(z, att_ref[m].reshape(HID, 1),
                    preferred_element_type=jnp.float32))
        outs.append(jnp.dot(z * a, c_ref[m],
                            preferred_element_type=jnp.float32))
    hcat = jax.nn.relu(jnp.concatenate(outs, axis=1))
    logits = jnp.dot(hcat, dw_ref[...],
                     preferred_element_type=jnp.float32) + db_ref[...]
    mx = jnp.max(logits, axis=1, keepdims=True)
    lse = mx + jnp.log(jnp.sum(jnp.exp(logits - mx), axis=1, keepdims=True))
    out_ref[...] = logits - lse


def _tc_dense(gp, W, att, C, dense_W, dense_b):
    return pl.pallas_call(
        _tc_dense_body,
        grid=(N // NB,),
        in_specs=[
            pl.BlockSpec((2, NC, M, NB, HH), lambda i: (0, 0, 0, i, 0)),
            pl.BlockSpec((M, HID, HID), lambda i: (0, 0, 0)),
            pl.BlockSpec((M, HID), lambda i: (0, 0)),
            pl.BlockSpec((M, HID, COMP), lambda i: (0, 0, 0)),
            pl.BlockSpec((M * COMP, CLS), lambda i: (0, 0)),
            pl.BlockSpec((CLS,), lambda i: (0,)),
        ],
        out_specs=pl.BlockSpec((NB, CLS), lambda i: (i, 0)),
        out_shape=jax.ShapeDtypeStruct((N, CLS), jnp.float32),
    )(gp, W, att, C, dense_W, dense_b)


def kernel(h, edge_index, motif_w, W, att, C, dense_W, dense_b):
    src3d = edge_index[0].reshape(NW, CPW, K)
    dst3d = edge_index[1].reshape(NW, CPW, K)
    w4d = motif_w.reshape(M, NW, CPW, K)
    gp = _sc_segment_sums(h[:, :HH], h[:, HH:], src3d, dst3d, w4d)
    return _tc_dense(gp, W, att, C, dense_W, dense_b)


# speedup vs baseline: 6.7058x; 1.0188x over previous
import functools

import jax
import jax.numpy as jnp
from jax import lax
from jax.experimental import pallas as pl
from jax.experimental.pallas import tpu as pltpu
from jax.experimental.pallas import tpu_sc as plsc

N = 10000
E = 320000
F_IN = 128
HID = 128
M = 13
COMP = 6
CLS = 7

NC = 2
NS = 16
NW = NC * NS
K = 80
CPW = E // K // NW
HH = HID // 2
ACH = 1000
ZR = 200


def _sc_segment_sums(h_lo, h_hi, src3d, dst3d, w4d):
    mesh = plsc.VectorSubcoreMesh(core_axis_name="c", subcore_axis_name="s")

    @functools.partial(
        pl.kernel,
        out_type=jax.ShapeDtypeStruct((2, NC, M, N, HH), jnp.float32),
        mesh=mesh,
        scratch_types=[
            pltpu.VMEM((CPW, K), jnp.int32),
            pltpu.VMEM((CPW, K), jnp.int32),
            pltpu.VMEM((CPW, K), jnp.float32),
            pltpu.VMEM((K, HH), jnp.float32),
            pltpu.VMEM((K, HH), jnp.float32),
            pltpu.VMEM((K, HH), jnp.float32),
            pltpu.VMEM((K, HH), jnp.float32),
            pltpu.VMEM((K, HH), jnp.float32),
            pltpu.VMEM((K, HH), jnp.float32),
            pltpu.VMEM((K, HH), jnp.float32),
            pltpu.VMEM((K, HH), jnp.float32),
            pltpu.VMEM((ZR, HH), jnp.float32),
            pltpu.VMEM_SHARED((N, HH), jnp.float32),
            pltpu.SemaphoreType.DMA,
            pltpu.SemaphoreType.DMA,
            pltpu.SemaphoreType.DMA,
            pltpu.SemaphoreType.DMA,
            pltpu.SemaphoreType.DMA,
            pltpu.SemaphoreType.DMA,
            pltpu.SemaphoreType.DMA,
            pltpu.SemaphoreType.DMA,
        ],
        compiler_params=pltpu.CompilerParams(use_tc_tiling_on_sc=False),
    )
    def body(hlo_hbm, hhi_hbm, src_hbm, dst_hbm, w_hbm, out_hbm,
             src_v, dst_v, w_v, rows0, rows1, rows2, rows3,
             sc0, sc1, sc2, sc3, zeros_v, acc,
             g0, g1, g2, g3, s0, s1, s2, s3):
        rows_bufs = (rows0, rows1, rows2, rows3)
        sc_bufs = (sc0, sc1, sc2, sc3)
        gsems = (g0, g1, g2, g3)
        ssems = (s0, s1, s2, s3)
        NBUF = 4
        cid = lax.axis_index("c")
        sid = lax.axis_index("s")
        wid = cid * NS + sid

        pltpu.sync_copy(src_hbm.at[wid], src_v)
        pltpu.sync_copy(dst_hbm.at[wid], dst_v)

        def zero_row(r, carry):
            for b in range(HH // 16):
                zeros_v[r, pl.ds(b * 16, 16)] = jnp.zeros((16,), jnp.float32)
            return carry

        lax.fori_loop(0, ZR, zero_row, 0)

        def scale(rows, scb, j):
            def grp_body(g, carry3):
                wv = w_v[j, pl.ds(g * 16, 16)]
                for e in range(16):
                    ws = wv[e]
                    row = g * 16 + e
                    for b in range(HH // 16):
                        sl = pl.ds(b * 16, 16)
                        scb[row, sl] = rows[row, sl] * ws
                return carry3
            lax.fori_loop(0, K // 16, grp_body, 0)

        def motif_body(m, carry):
            pltpu.sync_copy(w_hbm.at[m, wid], w_v)
            for hf, h_hbm in enumerate((hlo_hbm, hhi_hbm)):
                for b in range(NBUF):
                    pltpu.async_copy(h_hbm.at[src_v.at[b]], rows_bufs[b],
                                     gsems[b])

                @pl.when(sid < N // ACH)
                def _():
                    for z in range(ACH // ZR):
                        pltpu.sync_copy(
                            zeros_v, acc.at[pl.ds(sid * ACH + z * ZR, ZR)])
                plsc.subcore_barrier()

                def step(j, jj, bi):
                    rows, scb = rows_bufs[bi], sc_bufs[bi]
                    pltpu.make_async_copy(h_hbm.at[src_v.at[j]], rows,
                                          gsems[bi]).wait()

                    @pl.when(jj > 0)
                    def _():
                        pltpu.make_async_copy(scb, acc.at[dst_v.at[j]],
                                              ssems[bi]).wait()

                    scale(rows, scb, j)

                    @pl.when(j + NBUF < CPW)
                    def _():
                        pltpu.async_copy(h_hbm.at[src_v.at[j + NBUF]], rows,
                                         gsems[bi])

                    pltpu.async_copy(scb, acc.at[dst_v.at[j]], ssems[bi],
                                     add=True)

                def quad_body(jj, carry2):
                    for b in range(NBUF):
                        step(NBUF * jj + b, jj, b)
                    return carry2

                lax.fori_loop(0, CPW // NBUF, quad_body, 0)

                for t in range(CPW % NBUF):
                    j = (CPW // NBUF) * NBUF + t
                    rows, scb = rows_bufs[t], sc_bufs[t]
                    pltpu.make_async_copy(h_hbm.at[src_v.at[j]], rows,
                                          gsems[t]).wait()
                    pltpu.make_async_copy(scb, acc.at[dst_v.at[j]],
                                          ssems[t]).wait()
                    scale(rows, scb, j)
                    pltpu.async_copy(scb, acc.at[dst_v.at[j]], ssems[t],
                                     add=True)

                for b in range(NBUF):
                    pltpu.make_async_copy(sc_bufs[b], acc.at[dst_v.at[b]],
                                          ssems[b]).wait()
                plsc.subcore_barrier()

                @pl.when(sid < N // ACH)
                def _():
                    sl = pl.ds(sid * ACH, ACH)
                    pltpu.sync_copy(acc.at[sl], out_hbm.at[hf, cid, m, sl])
            return carry

        lax.fori_loop(0, M, motif_body, 0)

    return body(h_lo, h_hi, src3d, dst3d, w4d)


NB = 400


def _tc_dense_body(gp_ref, w_ref, att_ref, c_ref, dw_ref, db_ref, out_ref):
    outs = []
    for m in range(M):
        g = jnp.concatenate(
            [gp_ref[0, 0, m] + gp_ref[0, 1, m],
             gp_ref[1, 0, m] + gp_ref[1, 1, m]], axis=1)
        z = jnp.dot(g, w_ref[m], preferred_element_type=jnp.float32)
        a = jax.nn.sigmoid(
            jnp.dot(z, att_ref[m].reshape(HID, 1),
                    preferred_element_type=jnp.float32))
        outs.append(jnp.dot(z * a, c_ref[m],
                            preferred_element_type=jnp.float32))
    hcat = jax.nn.relu(jnp.concatenate(outs, axis=1))
    logits = jnp.dot(hcat, dw_ref[...],
                     preferred_element_type=jnp.float32) + db_ref[...]
    mx = jnp.max(logits, axis=1, keepdims=True)
    lse = mx + jnp.log(jnp.sum(jnp.exp(logits - mx), axis=1, keepdims=True))
    out_ref[...] = logits - lse


def _tc_dense(gp, W, att, C, dense_W, dense_b):
    return pl.pallas_call(
        _tc_dense_body,
        grid=(N // NB,),
        in_specs=[
            pl.BlockSpec((2, NC, M, NB, HH), lambda i: (0, 0, 0, i, 0)),
            pl.BlockSpec((M, HID, HID), lambda i: (0, 0, 0)),
            pl.BlockSpec((M, HID), lambda i: (0, 0)),
            pl.BlockSpec((M, HID, COMP), lambda i: (0, 0, 0)),
            pl.BlockSpec((M * COMP, CLS), lambda i: (0, 0)),
            pl.BlockSpec((CLS,), lambda i: (0,)),
        ],
        out_specs=pl.BlockSpec((NB, CLS), lambda i: (i, 0)),
        out_shape=jax.ShapeDtypeStruct((N, CLS), jnp.float32),
    )(gp, W, att, C, dense_W, dense_b)


def kernel(h, edge_index, motif_w, W, att, C, dense_W, dense_b):
    src3d = edge_index[0].reshape(NW, CPW, K)
    dst3d = edge_index[1].reshape(NW, CPW, K)
    w4d = motif_w.reshape(M, NW, CPW, K)
    gp = _sc_segment_sums(h[:, :HH], h[:, HH:], src3d, dst3d, w4d)
    return _tc_dense(gp, W, att, C, dense_W, dense_b)
